# Initial kernel scaffold; baseline (speedup 1.0000x reference)
import jax
import jax.numpy as jnp
from jax.experimental import pallas as pl


def kernel(edge_index, node_feats, W1, b1, W2, b2):
    raise NotImplementedError("write your pallas kernel here")


# speedup vs baseline: 6.3714x; 6.3714x over previous
import functools

import jax
import jax.numpy as jnp
from jax import lax
from jax.experimental import pallas as pl
from jax.experimental.pallas import tpu as pltpu
from jax.experimental.pallas import tpu_sc as plsc

N_NODES = 10000
N_EDGES = 320000
D = 128

NC = 2
NS = 16
NW = NC * NS
EPW = N_EDGES // NW
CHUNK = 80
ITERS = EPW // CHUNK
ROWS_PER_SUB = 624
TAIL_BASE = NS * ROWS_PER_SUB
TAIL = N_NODES - TAIL_BASE

_mesh = plsc.VectorSubcoreMesh(core_axis_name="c", subcore_axis_name="s")


@functools.partial(
    pl.kernel,
    mesh=_mesh,
    out_type=jax.ShapeDtypeStruct((2, N_NODES, D), jnp.float32),
    scratch_types=[
        pltpu.VMEM((ITERS, CHUNK), jnp.int32),
        pltpu.VMEM((ITERS, CHUNK), jnp.int32),
        pltpu.VMEM((CHUNK, D), jnp.float32),
        pltpu.VMEM_SHARED((N_NODES, D), jnp.float32),
        pltpu.SemaphoreType.DMA,
    ],
)
def _edge_agg(src_hbm, dst_hbm, h_hbm, out_hbm, sidx, didx, rows, acc, sem):
    cid = lax.axis_index("c")
    sid = lax.axis_index("s")
    wid = cid * NS + sid

    pltpu.sync_copy(src_hbm.at[wid], sidx)
    pltpu.sync_copy(dst_hbm.at[wid], didx)

    base = sid * ROWS_PER_SUB
    pltpu.sync_copy(h_hbm.at[pl.ds(base, ROWS_PER_SUB)],
                    acc.at[pl.ds(base, ROWS_PER_SUB)])

    @pl.when(sid == NS - 1)
    def _():
        pltpu.sync_copy(h_hbm.at[pl.ds(TAIL_BASE, TAIL)],
                        acc.at[pl.ds(TAIL_BASE, TAIL)])

    plsc.subcore_barrier()

    def body(i, carry):
        pltpu.async_copy(h_hbm.at[sidx.at[i]], rows, sem).wait()
        pltpu.sync_copy(rows, acc.at[didx.at[i]], add=True)
        return carry

    lax.fori_loop(0, ITERS, body, 0)

    plsc.subcore_barrier()
    pltpu.sync_copy(acc.at[pl.ds(base, ROWS_PER_SUB)],
                    out_hbm.at[cid, pl.ds(base, ROWS_PER_SUB)])

    @pl.when(sid == NS - 1)
    def _():
        pltpu.sync_copy(acc.at[pl.ds(TAIL_BASE, TAIL)],
                        out_hbm.at[cid, pl.ds(TAIL_BASE, TAIL)])


_BLK = 1000
_GRID = N_NODES // _BLK


def _mm(x, W, b):
    def body(x_ref, w_ref, b_ref, o_ref):
        o_ref[...] = jnp.dot(x_ref[...], w_ref[...],
                             preferred_element_type=jnp.float32) + b_ref[...]

    return pl.pallas_call(
        body,
        grid=(_GRID,),
        in_specs=[pl.BlockSpec((_BLK, D), lambda i: (i, 0)),
                  pl.BlockSpec((D, D), lambda i: (0, 0)),
                  pl.BlockSpec((1, D), lambda i: (0, 0))],
        out_specs=pl.BlockSpec((_BLK, D), lambda i: (i, 0)),
        out_shape=jax.ShapeDtypeStruct((N_NODES, D), jnp.float32),
    )(x, W, b.reshape(1, D))


def _combine_mm(p0, p1, h, W, b):
    def body(p0_ref, p1_ref, h_ref, w_ref, b_ref, o_ref):
        z = jnp.maximum(p0_ref[...] + p1_ref[...] - h_ref[...], 0.0)
        o_ref[...] = jnp.dot(z, w_ref[...],
                             preferred_element_type=jnp.float32) + b_ref[...]

    return pl.pallas_call(
        body,
        grid=(_GRID,),
        in_specs=[pl.BlockSpec((_BLK, D), lambda i: (i, 0)),
                  pl.BlockSpec((_BLK, D), lambda i: (i, 0)),
                  pl.BlockSpec((_BLK, D), lambda i: (i, 0)),
                  pl.BlockSpec((D, D), lambda i: (0, 0)),
                  pl.BlockSpec((1, D), lambda i: (0, 0))],
        out_specs=pl.BlockSpec((_BLK, D), lambda i: (i, 0)),
        out_shape=jax.ShapeDtypeStruct((N_NODES, D), jnp.float32),
    )(p0, p1, h, W, b.reshape(1, D))


def _combine_relu(p0, p1, h):
    def body(p0_ref, p1_ref, h_ref, o_ref):
        o_ref[...] = jnp.maximum(p0_ref[...] + p1_ref[...] - h_ref[...], 0.0)

    return pl.pallas_call(
        body,
        grid=(_GRID,),
        in_specs=[pl.BlockSpec((_BLK, D), lambda i: (i, 0)),
                  pl.BlockSpec((_BLK, D), lambda i: (i, 0)),
                  pl.BlockSpec((_BLK, D), lambda i: (i, 0))],
        out_specs=pl.BlockSpec((_BLK, D), lambda i: (i, 0)),
        out_shape=jax.ShapeDtypeStruct((N_NODES, D), jnp.float32),
    )(p0, p1, h)


def kernel(edge_index, node_feats, W1, b1, W2, b2):
    src = edge_index[0].astype(jnp.int32).reshape(NW, ITERS, CHUNK)
    dst = edge_index[1].astype(jnp.int32).reshape(NW, ITERS, CHUNK)
    h1 = _mm(node_feats, W1, b1)
    p = _edge_agg(src, dst, h1)
    h2 = _combine_mm(p[0], p[1], h1, W2, b2)
    q = _edge_agg(src, dst, h2)
    return _combine_relu(q[0], q[1], h2)


# speedup vs baseline: 9.9167x; 1.5564x over previous
import functools

import jax
import jax.numpy as jnp
from jax import lax
from jax.experimental import pallas as pl
from jax.experimental.pallas import tpu as pltpu
from jax.experimental.pallas import tpu_sc as plsc

N_NODES = 10000
N_EDGES = 320000
D = 128

NC = 2
NS = 16
NW = NC * NS
EPW = N_EDGES // NW
CHUNK = 80
ITERS = EPW // CHUNK
ROWS_PER_SUB = 624
TAIL_BASE = NS * ROWS_PER_SUB
TAIL = N_NODES - TAIL_BASE

_mesh = plsc.VectorSubcoreMesh(core_axis_name="c", subcore_axis_name="s")


@functools.partial(
    pl.kernel,
    mesh=_mesh,
    out_type=jax.ShapeDtypeStruct((2, N_NODES, D), jnp.float32),
    scratch_types=[
        pltpu.VMEM((EPW,), jnp.int32),
        pltpu.VMEM((ITERS, CHUNK), jnp.int32),
        pltpu.VMEM((CHUNK, D), jnp.float32),
        pltpu.VMEM((CHUNK, D), jnp.float32),
        pltpu.VMEM_SHARED((N_NODES, D), jnp.float32),
        pltpu.SemaphoreType.DMA,
        pltpu.SemaphoreType.DMA,
    ],
)
def _edge_agg(src_hbm, dst_hbm, h_hbm, out_hbm, sidx, didx, rows0, rows1,
              acc, sem0, sem1):
    cid = lax.axis_index("c")
    sid = lax.axis_index("s")
    wid = cid * NS + sid

    pltpu.sync_copy(src_hbm.at[pl.ds(wid * EPW, EPW)], sidx)
    pltpu.sync_copy(dst_hbm.at[wid], didx)

    base = sid * ROWS_PER_SUB
    pltpu.sync_copy(h_hbm.at[pl.ds(base, ROWS_PER_SUB)],
                    acc.at[pl.ds(base, ROWS_PER_SUB)])

    @pl.when(sid == NS - 1)
    def _():
        pltpu.sync_copy(h_hbm.at[pl.ds(TAIL_BASE, TAIL)],
                        acc.at[pl.ds(TAIL_BASE, TAIL)])

    plsc.subcore_barrier()

    def _sidx_chunk(i):
        return sidx.at[pl.ds(pl.multiple_of(i * CHUNK, 8), CHUNK)]

    pltpu.async_copy(h_hbm.at[_sidx_chunk(0)], rows0, sem0)

    def body(j, carry):
        i0 = 2 * j
        pltpu.async_copy(h_hbm.at[_sidx_chunk(i0 + 1)], rows1, sem1)
        pltpu.make_async_copy(h_hbm.at[_sidx_chunk(i0)], rows0, sem0).wait()
        pltpu.sync_copy(rows0, acc.at[didx.at[i0]], add=True)
        pltpu.async_copy(h_hbm.at[_sidx_chunk(i0 + 2)], rows0, sem0)
        pltpu.make_async_copy(h_hbm.at[_sidx_chunk(i0 + 1)], rows1, sem1).wait()
        pltpu.sync_copy(rows1, acc.at[didx.at[i0 + 1]], add=True)
        return carry

    lax.fori_loop(0, (ITERS - 1) // 2, body, 0)
    pltpu.make_async_copy(h_hbm.at[_sidx_chunk(ITERS - 1)], rows0, sem0).wait()
    pltpu.sync_copy(rows0, acc.at[didx.at[ITERS - 1]], add=True)

    plsc.subcore_barrier()
    pltpu.sync_copy(acc.at[pl.ds(base, ROWS_PER_SUB)],
                    out_hbm.at[cid, pl.ds(base, ROWS_PER_SUB)])

    @pl.when(sid == NS - 1)
    def _():
        pltpu.sync_copy(acc.at[pl.ds(TAIL_BASE, TAIL)],
                        out_hbm.at[cid, pl.ds(TAIL_BASE, TAIL)])


_BLK = 1000
_GRID = N_NODES // _BLK


def _mm(x, W, b):
    def body(x_ref, w_ref, b_ref, o_ref):
        o_ref[...] = jnp.dot(x_ref[...], w_ref[...],
                             preferred_element_type=jnp.float32) + b_ref[...]

    return pl.pallas_call(
        body,
        grid=(_GRID,),
        in_specs=[pl.BlockSpec((_BLK, D), lambda i: (i, 0)),
                  pl.BlockSpec((D, D), lambda i: (0, 0)),
                  pl.BlockSpec((1, D), lambda i: (0, 0))],
        out_specs=pl.BlockSpec((_BLK, D), lambda i: (i, 0)),
        out_shape=jax.ShapeDtypeStruct((N_NODES, D), jnp.float32),
    )(x, W, b.reshape(1, D))


def _combine_mm(p0, p1, h, W, b):
    def body(p0_ref, p1_ref, h_ref, w_ref, b_ref, o_ref):
        z = jnp.maximum(p0_ref[...] + p1_ref[...] - h_ref[...], 0.0)
        o_ref[...] = jnp.dot(z, w_ref[...],
                             preferred_element_type=jnp.float32) + b_ref[...]

    return pl.pallas_call(
        body,
        grid=(_GRID,),
        in_specs=[pl.BlockSpec((_BLK, D), lambda i: (i, 0)),
                  pl.BlockSpec((_BLK, D), lambda i: (i, 0)),
                  pl.BlockSpec((_BLK, D), lambda i: (i, 0)),
                  pl.BlockSpec((D, D), lambda i: (0, 0)),
                  pl.BlockSpec((1, D), lambda i: (0, 0))],
        out_specs=pl.BlockSpec((_BLK, D), lambda i: (i, 0)),
        out_shape=jax.ShapeDtypeStruct((N_NODES, D), jnp.float32),
    )(p0, p1, h, W, b.reshape(1, D))


def _combine_relu(p0, p1, h):
    def body(p0_ref, p1_ref, h_ref, o_ref):
        o_ref[...] = jnp.maximum(p0_ref[...] + p1_ref[...] - h_ref[...], 0.0)

    return pl.pallas_call(
        body,
        grid=(_GRID,),
        in_specs=[pl.BlockSpec((_BLK, D), lambda i: (i, 0)),
                  pl.BlockSpec((_BLK, D), lambda i: (i, 0)),
                  pl.BlockSpec((_BLK, D), lambda i: (i, 0))],
        out_specs=pl.BlockSpec((_BLK, D), lambda i: (i, 0)),
        out_shape=jax.ShapeDtypeStruct((N_NODES, D), jnp.float32),
    )(p0, p1, h)


def kernel(edge_index, node_feats, W1, b1, W2, b2):
    src = edge_index[0].astype(jnp.int32)
    dst = edge_index[1].astype(jnp.int32).reshape(NW, ITERS, CHUNK)
    h1 = _mm(node_feats, W1, b1)
    p = _edge_agg(src, dst, h1)
    h2 = _combine_mm(p[0], p[1], h1, W2, b2)
    q = _edge_agg(src, dst, h2)
    return _combine_relu(q[0], q[1], h2)
